# baseline (speedup 1.0000x reference)
import jax
import jax.numpy as jnp
from jax import lax
from jax.experimental import pallas as pl
from jax.experimental.pallas import tpu as pltpu
from jax.experimental.pallas import tpu_sc as plsc

_NC = 2
_NS = 16
_L = 16
_NW = _NC * _NS

_V = 1000000
_B = 16384
_BPW = _B // _NW
_DA = 63
_NG = 5
_CHUNK = 128
_NCHUNK = _BPW // _CHUNK
_NGROUP = _BPW // _L
_GPC = _CHUNK // _L

_WG_A = _V * _DA // _L
_WG_S = _V // _L


def _glove_body(t_idx_hbm, c_idx_hbm, ta_hbm, tg_hbm, tb_hbm,
                ca_hbm, cg_hbm, cb_hbm, out_hbm,
                idx_t, idx_c, gi_t, gi_c, si_t, si_c, ta_v, ca_v,
                tg_v, cg_v, tb_v, cb_v, out_v, sem):
  wid = lax.axis_index("s") * _NC + lax.axis_index("c")
  base = wid * _BPW
  lane = lax.iota(jnp.int32, _L)

  for c in range(_NCHUNK):
    src = pl.ds(base + c * _CHUNK, _CHUNK)
    pltpu.sync_copy(t_idx_hbm.at[src], idx_t.at[c])
    pltpu.sync_copy(c_idx_hbm.at[src], idx_c.at[c])

  for c in range(_NCHUNK):
    for g in range(_GPC):
      sl = pl.ds(g * _L, _L)
      vt = idx_t[c, sl]
      vc = idx_c[c, sl]
      r0t = (vt * _DA) >> 4
      r0c = (vc * _DA) >> 4
      for t in range(_NG):
        gi_t[c * _NG + t, sl] = r0t + t
        gi_c[c * _NG + t, sl] = r0c + t
      si_t[c, sl] = vt >> 4
      si_c[c, sl] = vc >> 4

  copies = []
  for c in range(_NCHUNK):
    for t in range(_NG):
      r = c * _NG + t
      dst = pl.ds(r * _CHUNK, _CHUNK)
      copies.append(pltpu.async_copy(ta_hbm.at[gi_t.at[r]], ta_v.at[dst], sem))
      copies.append(pltpu.async_copy(ca_hbm.at[gi_c.at[r]], ca_v.at[dst], sem))
    dst = pl.ds(c * _CHUNK, _CHUNK)
    copies.append(pltpu.async_copy(tg_hbm.at[si_t.at[c]], tg_v.at[dst], sem))
    copies.append(pltpu.async_copy(cg_hbm.at[si_c.at[c]], cg_v.at[dst], sem))
    copies.append(pltpu.async_copy(tb_hbm.at[si_t.at[c]], tb_v.at[dst], sem))
    copies.append(pltpu.async_copy(cb_hbm.at[si_c.at[c]], cb_v.at[dst], sem))
  for cp in copies:
    cp.wait()

  def group(g, carry):
    c = g >> 3
    gg = g & 7
    sl = pl.ds(gg * _L, _L)
    vt = idx_t[c, sl]
    vc = idx_c[c, sl]
    erow = gg * _L + lane
    srow = c * _CHUNK + erow
    tg = plsc.load_gather(tg_v, [srow, vt & 15])
    cg = plsc.load_gather(cg_v, [srow, vc & 15])
    tb = plsc.load_gather(tb_v, [srow, vt & 15])
    cb = plsc.load_gather(cb_v, [srow, vc & 15])
    acc = tb + cb + tg * cg
    base_t = (vt * _DA) & 15
    base_c = (vc * _DA) & 15
    brow = c * (_NG * _CHUNK) + erow
    for k in range(_DA):
      bt = base_t + k
      bc = base_c + k
      row_t = brow + (bt >> 4) * _CHUNK
      row_c = brow + (bc >> 4) * _CHUNK
      tv = plsc.load_gather(ta_v, [row_t, bt & 15])
      cv = plsc.load_gather(ca_v, [row_c, bc & 15])
      acc = acc + tv * cv
    out_v[pl.ds(g * _L, _L)] = acc
    return carry

  lax.fori_loop(0, _NGROUP, group, 0)

  pltpu.sync_copy(out_v, out_hbm.at[pl.ds(base, _BPW)])


@jax.jit
def _glove_sc(t_idx, c_idx, ta, tg, tb, ca, cg, cb):
  mesh = plsc.VectorSubcoreMesh(core_axis_name="c", subcore_axis_name="s",
                                num_cores=_NC, num_subcores=_NS)
  f = pl.kernel(
      _glove_body,
      out_type=jax.ShapeDtypeStruct((_B,), jnp.float32),
      mesh=mesh,
      compiler_params=pltpu.CompilerParams(needs_layout_passes=False,
                                           use_tc_tiling_on_sc=False),
      scratch_types=[
          pltpu.VMEM((_NCHUNK, _CHUNK), jnp.int32),
          pltpu.VMEM((_NCHUNK, _CHUNK), jnp.int32),
          pltpu.VMEM((_NCHUNK * _NG, _CHUNK), jnp.int32),
          pltpu.VMEM((_NCHUNK * _NG, _CHUNK), jnp.int32),
          pltpu.VMEM((_NCHUNK, _CHUNK), jnp.int32),
          pltpu.VMEM((_NCHUNK, _CHUNK), jnp.int32),
          pltpu.VMEM((_NCHUNK * _NG * _CHUNK, _L), jnp.float32),
          pltpu.VMEM((_NCHUNK * _NG * _CHUNK, _L), jnp.float32),
          pltpu.VMEM((_BPW, _L), jnp.float32),
          pltpu.VMEM((_BPW, _L), jnp.float32),
          pltpu.VMEM((_BPW, _L), jnp.float32),
          pltpu.VMEM((_BPW, _L), jnp.float32),
          pltpu.VMEM((_BPW,), jnp.float32),
          pltpu.SemaphoreType.DMA,
      ],
  )
  return f(t_idx, c_idx, ta, tg, tb, ca, cg, cb)


def kernel(target_idx, context_idx, target_emb_a, target_emb_g, target_bias,
           context_emb_a, context_emb_g, context_bias):
  out = _glove_sc(target_idx, context_idx,
                  target_emb_a.reshape(_WG_A, _L),
                  target_emb_g.reshape(_WG_S, _L),
                  target_bias.reshape(_WG_S, _L),
                  context_emb_a.reshape(_WG_A, _L),
                  context_emb_g.reshape(_WG_S, _L),
                  context_bias.reshape(_WG_S, _L))
  return out.reshape(_B, 1)


# speedup vs baseline: 1.4450x; 1.4450x over previous
import jax
import jax.numpy as jnp
from jax import lax
from jax.experimental import pallas as pl
from jax.experimental.pallas import tpu as pltpu
from jax.experimental.pallas import tpu_sc as plsc

_NC = 2
_NS = 16
_L = 16
_NW = _NC * _NS

_V = 1000000
_B = 16384
_BPW = _B // _NW
_DA = 63
_CHUNK = 128
_NCHUNK = _BPW // _CHUNK
_NGROUP = _BPW // _L
_GPC = _CHUNK // _L
_WG_S = _V // _L


def _dot_body(t_idx_hbm, c_idx_hbm, ta_hbm, ca_hbm, out_hbm,
              it_v, ic_v, t0, c0, t1, c1, ov,
              s0t, s0c, s1t, s1c, semi):
  wid = lax.axis_index("s") * _NC + lax.axis_index("c")
  base = wid * _BPW
  lane = lax.iota(jnp.int32, _L)

  pltpu.async_copy(t_idx_hbm.at[pl.ds(base, _BPW)], it_v, semi).wait()
  pltpu.async_copy(c_idx_hbm.at[pl.ds(base, _BPW)], ic_v, semi).wait()

  def fetch(g, buf_t, buf_c, st, sc):
    off = g * _L
    vt = it_v[pl.ds(off, _L)]
    vc = ic_v[pl.ds(off, _L)]
    for e in range(_L):
      v8t = pl.multiple_of((vt[e] >> 3) << 3, 8)
      v8c = pl.multiple_of((vc[e] >> 3) << 3, 8)
      pltpu.async_copy(ta_hbm.at[pl.ds(v8t, 8)], buf_t.at[pl.ds(e * 8, 8)], st)
      pltpu.async_copy(ca_hbm.at[pl.ds(v8c, 8)], buf_c.at[pl.ds(e * 8, 8)], sc)

  def waitbuf(buf_t, buf_c, st, sc):
    pltpu.make_async_copy(ta_hbm.at[pl.ds(0, 8 * _L)], buf_t, st).wait()
    pltpu.make_async_copy(ca_hbm.at[pl.ds(0, 8 * _L)], buf_c, sc).wait()

  def compute(g, buf_t, buf_c):
    off = g * _L
    vt = it_v[pl.ds(off, _L)]
    vc = ic_v[pl.ds(off, _L)]
    rt = lane * 8 + (vt & 7)
    rc = lane * 8 + (vc & 7)
    acc = jnp.zeros((_L,), jnp.float32)
    for k in range(_DA):
      kv = jnp.full((_L,), k, jnp.int32)
      acc = acc + plsc.load_gather(buf_t, [rt, kv]) * plsc.load_gather(buf_c, [rc, kv])
    ov[pl.ds(off, _L)] = acc

  fetch(0, t0, c0, s0t, s0c)

  def body(i, carry):
    g0 = 2 * i
    g1 = 2 * i + 1
    fetch(g1, t1, c1, s1t, s1c)
    waitbuf(t0, c0, s0t, s0c)
    compute(g0, t0, c0)
    fetch(jnp.minimum(g1 + 1, _NGROUP - 1), t0, c0, s0t, s0c)
    waitbuf(t1, c1, s1t, s1c)
    compute(g1, t1, c1)
    return carry

  lax.fori_loop(0, _NGROUP // 2, body, 0)
  waitbuf(t0, c0, s0t, s0c)

  pltpu.sync_copy(ov, out_hbm.at[pl.ds(base, _BPW)])


def _small_body(t_idx_hbm, c_idx_hbm, tg_hbm, tb_hbm, cg_hbm, cb_hbm, out_hbm,
                idx_t, idx_c, si_t, si_c, tg_v, cg_v, tb_v, cb_v, out_v, sem):
  wid = lax.axis_index("s") * _NC + lax.axis_index("c")
  base = wid * _BPW
  lane = lax.iota(jnp.int32, _L)

  for c in range(_NCHUNK):
    src = pl.ds(base + c * _CHUNK, _CHUNK)
    pltpu.sync_copy(t_idx_hbm.at[src], idx_t.at[c])
    pltpu.sync_copy(c_idx_hbm.at[src], idx_c.at[c])

  for c in range(_NCHUNK):
    for g in range(_GPC):
      sl = pl.ds(g * _L, _L)
      si_t[c, sl] = idx_t[c, sl] >> 4
      si_c[c, sl] = idx_c[c, sl] >> 4

  copies = []
  for c in range(_NCHUNK):
    dst = pl.ds(c * _CHUNK, _CHUNK)
    copies.append(pltpu.async_copy(tg_hbm.at[si_t.at[c]], tg_v.at[dst], sem))
    copies.append(pltpu.async_copy(cg_hbm.at[si_c.at[c]], cg_v.at[dst], sem))
    copies.append(pltpu.async_copy(tb_hbm.at[si_t.at[c]], tb_v.at[dst], sem))
    copies.append(pltpu.async_copy(cb_hbm.at[si_c.at[c]], cb_v.at[dst], sem))
  for cp in copies:
    cp.wait()

  def group(g, carry):
    c = g >> 3
    gg = g & 7
    sl = pl.ds(gg * _L, _L)
    vt = idx_t[c, sl]
    vc = idx_c[c, sl]
    srow = c * _CHUNK + gg * _L + lane
    tg = plsc.load_gather(tg_v, [srow, vt & 15])
    cg = plsc.load_gather(cg_v, [srow, vc & 15])
    tb = plsc.load_gather(tb_v, [srow, vt & 15])
    cb = plsc.load_gather(cb_v, [srow, vc & 15])
    out_v[pl.ds(g * _L, _L)] = tb + cb + tg * cg
    return carry

  lax.fori_loop(0, _NGROUP, group, 0)

  pltpu.sync_copy(out_v, out_hbm.at[pl.ds(base, _BPW)])


@jax.jit
def _glove_sc(t_idx, c_idx, ta, tg, tb, ca, cg, cb):
  mesh = plsc.VectorSubcoreMesh(core_axis_name="c", subcore_axis_name="s",
                                num_cores=_NC, num_subcores=_NS)
  dot = pl.kernel(
      _dot_body,
      out_type=jax.ShapeDtypeStruct((_B,), jnp.float32),
      mesh=mesh,
      compiler_params=pltpu.CompilerParams(needs_layout_passes=False,
                                           use_tc_tiling_on_sc=True),
      scratch_types=[
          pltpu.VMEM((_BPW,), jnp.int32),
          pltpu.VMEM((_BPW,), jnp.int32),
          pltpu.VMEM((8 * _L, _DA), jnp.float32),
          pltpu.VMEM((8 * _L, _DA), jnp.float32),
          pltpu.VMEM((8 * _L, _DA), jnp.float32),
          pltpu.VMEM((8 * _L, _DA), jnp.float32),
          pltpu.VMEM((_BPW,), jnp.float32),
          pltpu.SemaphoreType.DMA,
          pltpu.SemaphoreType.DMA,
          pltpu.SemaphoreType.DMA,
          pltpu.SemaphoreType.DMA,
          pltpu.SemaphoreType.DMA,
      ],
  )
  small = pl.kernel(
      _small_body,
      out_type=jax.ShapeDtypeStruct((_B,), jnp.float32),
      mesh=mesh,
      compiler_params=pltpu.CompilerParams(needs_layout_passes=False,
                                           use_tc_tiling_on_sc=False),
      scratch_types=[
          pltpu.VMEM((_NCHUNK, _CHUNK), jnp.int32),
          pltpu.VMEM((_NCHUNK, _CHUNK), jnp.int32),
          pltpu.VMEM((_NCHUNK, _CHUNK), jnp.int32),
          pltpu.VMEM((_NCHUNK, _CHUNK), jnp.int32),
          pltpu.VMEM((_BPW, _L), jnp.float32),
          pltpu.VMEM((_BPW, _L), jnp.float32),
          pltpu.VMEM((_BPW, _L), jnp.float32),
          pltpu.VMEM((_BPW, _L), jnp.float32),
          pltpu.VMEM((_BPW,), jnp.float32),
          pltpu.SemaphoreType.DMA,
      ],
  )
  d = dot(t_idx, c_idx, ta, ca)
  s = small(t_idx, c_idx, tg, tb, cg, cb)
  return d + s


def kernel(target_idx, context_idx, target_emb_a, target_emb_g, target_bias,
           context_emb_a, context_emb_g, context_bias):
  out = _glove_sc(target_idx, context_idx,
                  target_emb_a,
                  target_emb_g.reshape(_WG_S, _L),
                  target_bias.reshape(_WG_S, _L),
                  context_emb_a,
                  context_emb_g.reshape(_WG_S, _L),
                  context_bias.reshape(_WG_S, _L))
  return out.reshape(_B, 1)
